# baseline (speedup 1.0000x reference)
import functools

import jax
import jax.numpy as jnp
from jax import lax
from jax.experimental import pallas as pl
from jax.experimental.pallas import tpu as pltpu
from jax.experimental.pallas import tpu_sc as plsc

B, N, K = 8, 2048, 16
BN = B * N
NH = N // 2


_RT = 256


def _knn_body(hr_ref, hf_ref, idx_ref):
    b = pl.program_id(0)
    hr = hr_ref[0]
    hf = hf_ref[0]
    sqf = jnp.sum(hf * hf, axis=1, keepdims=True)
    sqr = jnp.sum(hr * hr, axis=1, keepdims=True)
    g = lax.dot_general(hr, hf, (((1,), (1,)), ((), ())),
                        preferred_element_type=jnp.float32)
    d2 = sqr + sqf.T - 2.0 * g
    coliof = lax.broadcasted_iota(jnp.int32, (_RT, N), 1).astype(jnp.float32)
    iof = lax.broadcasted_iota(jnp.int32, (_RT, 128), 1).astype(jnp.float32)
    ng = N // 128
    cols = []
    for k in range(K):
        val = d2[:, 0:128]
        idxf = iof
        for gi in range(1, ng):
            v = d2[:, gi * 128:(gi + 1) * 128]
            cond = v < val
            val = jnp.minimum(val, v)
            idxf = jnp.where(cond, iof + (128.0 * gi), idxf)
        m = jnp.min(val, axis=1, keepdims=True)
        amf = jnp.min(jnp.where(val == m, idxf, 3.0e9), axis=1,
                      keepdims=True)
        cols.append(amf.astype(jnp.int32))
        if k < K - 1:
            d2 = jnp.where(coliof == amf, jnp.inf, d2)
    idx_ref[0] = jnp.concatenate(cols, axis=1) + b * N


def _knn_topk(h3, half):
    _, _, d = h3.shape
    nrt = NH // _RT
    grid = (B, nrt)
    return pl.pallas_call(
        _knn_body,
        grid=grid,
        in_specs=[
            pl.BlockSpec((1, _RT, d), lambda b, r: (b, r + half * nrt, 0)),
            pl.BlockSpec((1, N, d), lambda b, r: (b, 0, 0)),
        ],
        out_specs=pl.BlockSpec((1, _RT, K), lambda b, r: (b, r, 0)),
        out_shape=jax.ShapeDtypeStruct((B, NH, K), jnp.int32),
    )(h3, h3)


def _pad128_body(x_ref, out_ref):
    x = x_ref[...]
    out_ref[...] = jnp.concatenate(
        [x, jnp.zeros((x.shape[0], 128 - x.shape[1]), jnp.float32)], axis=1)


def _pad128(x2):
    M, d = x2.shape
    T = 1024
    return pl.pallas_call(
        _pad128_body,
        grid=(M // T,),
        in_specs=[pl.BlockSpec((T, d), lambda i: (i, 0))],
        out_specs=pl.BlockSpec((T, 128), lambda i: (i, 0)),
        out_shape=jax.ShapeDtypeStruct((M, 128), jnp.float32),
    )(x2)


_NC, _NS = 2, 16
_NW = _NC * _NS
_CH = 8


def _gather_h(idx_flat, hp):
    nidx = idx_flat.shape[0] // K
    npw = nidx // _NW
    nch = npw // _CH
    mesh = plsc.VectorSubcoreMesh(core_axis_name="c", subcore_axis_name="s")

    @functools.partial(
        pl.kernel,
        mesh=mesh,
        out_type=jax.ShapeDtypeStruct((nidx * K, 128), jnp.float32),
        scratch_types=[
            pltpu.VMEM((_CH * K,), jnp.int32),
            pltpu.VMEM((_CH * K,), jnp.int32),
            pltpu.VMEM((_CH * K, 128), jnp.float32),
            pltpu.VMEM((_CH * K, 128), jnp.float32),
            pltpu.SemaphoreType.DMA,
            pltpu.SemaphoreType.DMA,
        ],
    )
    def sc_k(idx_hbm, h_hbm, out_hbm,
             idx_v0, idx_v1, rows_v0, rows_v1, gs0, gs1):
        wid = lax.axis_index("s") * _NC + lax.axis_index("c")
        w0 = wid * npw * K
        idx_v = (idx_v0, idx_v1)
        rows_v = (rows_v0, rows_v1)
        gsem = (gs0, gs1)

        def start(ci, sl):
            base = w0 + ci * (_CH * K)
            pltpu.sync_copy(idx_hbm.at[pl.ds(base, _CH * K)], idx_v[sl])
            pltpu.async_copy(h_hbm.at[idx_v[sl]], rows_v[sl], gsem[sl])

        def drain(ci, sl):
            pltpu.make_async_copy(h_hbm.at[idx_v[sl]], rows_v[sl],
                                  gsem[sl]).wait()
            base = w0 + ci * (_CH * K)
            pltpu.sync_copy(rows_v[sl], out_hbm.at[pl.ds(base, _CH * K)])

        start(0, 0)

        def chunk_body(ci2, carry):
            c0 = ci2 * 2
            start(c0 + 1, 1)
            drain(c0, 0)

            @pl.when(c0 + 2 < nch)
            def _():
                start(c0 + 2, 0)

            drain(c0 + 1, 1)
            return carry

        lax.fori_loop(0, nch // 2, chunk_body, 0)

    return sc_k(idx_flat, hp)


_VT = 64


def _edge_conv_body(hnb_ref, h_ref, tw_ref, tb_ref, pw_ref, pb_ref,
                    maxe_ref, sums_ref, acc_ref):
    i = pl.program_id(0)
    j = pl.program_id(1)
    din = h_ref.shape[2]
    dout = tw_ref.shape[1]

    @pl.when((i == 0) & (j == 0))
    def _():
        acc_ref[...] = jnp.zeros_like(acc_ref)

    hn = hnb_ref[...][:, :din]
    hv = h_ref[0]
    hv_rep = jnp.broadcast_to(hv[:, None, :], (_VT, K, din))
    theta = hv_rep.reshape(_VT * K, din) - hn
    t1 = jnp.dot(theta, tw_ref[...], preferred_element_type=jnp.float32)
    phi = jnp.dot(hv, pw_ref[...],
                  preferred_element_type=jnp.float32) + pb_ref[0:1, :]
    e3 = (t1 + tb_ref[0:1, :]).reshape(_VT, K, dout) + phi[:, None, :]
    maxe_ref[0] = jnp.max(e3, axis=1)
    ef = e3.reshape(_VT * K, dout)
    acc_ref[0:1, :] += jnp.sum(ef, axis=0, keepdims=True)
    acc_ref[1:2, :] += jnp.sum(ef * ef, axis=0, keepdims=True)

    @pl.when((i == pl.num_programs(0) - 1) & (j == pl.num_programs(1) - 1))
    def _():
        sums_ref[...] = acc_ref[...]


def _edge_conv_max(hnb, h3, tw, tb, pw, pb, half):
    din, dout = tw.shape
    w = hnb.shape[1]
    nvt = NH // _VT
    grid = (B, nvt)
    return pl.pallas_call(
        _edge_conv_body,
        grid=grid,
        in_specs=[
            pl.BlockSpec((_VT * K, w), lambda b, j: (b * nvt + j, 0)),
            pl.BlockSpec((1, _VT, din),
                         lambda b, j: (b, j + half * nvt, 0)),
            pl.BlockSpec((din, dout), lambda b, j: (0, 0)),
            pl.BlockSpec((1, dout), lambda b, j: (0, 0)),
            pl.BlockSpec((din, dout), lambda b, j: (0, 0)),
            pl.BlockSpec((1, dout), lambda b, j: (0, 0)),
        ],
        out_specs=[
            pl.BlockSpec((1, _VT, dout), lambda b, j: (b, j, 0)),
            pl.BlockSpec((8, dout), lambda b, j: (0, 0)),
        ],
        out_shape=[
            jax.ShapeDtypeStruct((B, NH, dout), jnp.float32),
            jax.ShapeDtypeStruct((8, dout), jnp.float32),
        ],
        scratch_shapes=[pltpu.VMEM((8, dout), jnp.float32)],
    )(hnb, h3, tw, tb.reshape(1, dout), pw, pb.reshape(1, dout))


def _bn_final_body(x0_ref, x1_ref, s0_ref, s1_ref, g_ref, b_ref,
                   out_ref, pad_ref):
    E = float(BN * K)
    s0 = s0_ref[0:2, :] + s1_ref[0:2, :]
    mu = s0[0:1, :] / E
    var = jnp.maximum(s0[1:2, :] / E - mu * mu, 0.0)
    scale = g_ref[0:1, :]
    shift = b_ref[0:1, :]
    den = jnp.sqrt(var + 1e-5)

    def bn(x):
        t = (x - mu) / den * scale + shift
        return jnp.where(t >= 0, t, 0.2 * t)

    h0 = bn(x0_ref[0])
    h1 = bn(x1_ref[0])
    out_ref[0, :NH] = h0
    out_ref[0, NH:] = h1
    d = h0.shape[1]
    if d < 128:
        z = jnp.zeros((NH, 128 - d), jnp.float32)
        pad_ref[0, :NH] = jnp.concatenate([h0, z], axis=1)
        pad_ref[0, NH:] = jnp.concatenate([h1, z], axis=1)
    else:
        pad_ref[0, :NH] = h0
        pad_ref[0, NH:] = h1


def _bn_final(x0, x1, s0, s1, g, b):
    D = x0.shape[-1]
    return pl.pallas_call(
        _bn_final_body,
        grid=(B,),
        in_specs=[
            pl.BlockSpec((1, NH, D), lambda b: (b, 0, 0)),
            pl.BlockSpec((1, NH, D), lambda b: (b, 0, 0)),
            pl.BlockSpec((8, D), lambda b: (0, 0)),
            pl.BlockSpec((8, D), lambda b: (0, 0)),
            pl.BlockSpec((1, D), lambda b: (0, 0)),
            pl.BlockSpec((1, D), lambda b: (0, 0)),
        ],
        out_specs=[
            pl.BlockSpec((1, N, D), lambda b: (b, 0, 0)),
            pl.BlockSpec((1, N, 128), lambda b: (b, 0, 0)),
        ],
        out_shape=[
            jax.ShapeDtypeStruct((B, N, D), jnp.float32),
            jax.ShapeDtypeStruct((B, N, 128), jnp.float32),
        ],
    )(x0, x1, s0, s1, g.reshape(1, D), b.reshape(1, D))


def _pool_body(h_ref, gw_ref, gb_ref, fw_ref, fb_ref, out_ref):
    h = h_ref[0]
    g = jnp.dot(h, gw_ref[...],
                preferred_element_type=jnp.float32) + gb_ref[0:1, :]
    f = jnp.dot(h, fw_ref[...],
                preferred_element_type=jnp.float32) + fb_ref[0:1, :]
    g = jnp.maximum(g, 0.0)
    f = jnp.maximum(f, 0.0)
    mx = jnp.max(g, axis=0, keepdims=True)
    e = jnp.exp(g - mx)
    z = jnp.sum(e, axis=0, keepdims=True)
    out_ref[0] = jnp.sum(e * f, axis=0, keepdims=True) / z


def _attn_pool(h3, gw, gb, fw, fb):
    _, _, D = h3.shape
    DO = gw.shape[1]
    out = pl.pallas_call(
        _pool_body,
        grid=(B,),
        in_specs=[
            pl.BlockSpec((1, N, D), lambda b: (b, 0, 0)),
            pl.BlockSpec((D, DO), lambda b: (0, 0)),
            pl.BlockSpec((1, DO), lambda b: (0, 0)),
            pl.BlockSpec((D, DO), lambda b: (0, 0)),
            pl.BlockSpec((1, DO), lambda b: (0, 0)),
        ],
        out_specs=pl.BlockSpec((1, 1, DO), lambda b: (b, 0, 0)),
        out_shape=jax.ShapeDtypeStruct((B, 1, DO), jnp.float32),
    )(h3, gw, gb.reshape(1, DO), fw, fb.reshape(1, DO))
    return out.reshape(B, DO)


def kernel(x, theta_w0, theta_b0, phi_w0, phi_b0, bn_g0, bn_b0,
           theta_w1, theta_b1, phi_w1, phi_b1, bn_g1, bn_b1,
           theta_w2, theta_b2, phi_w2, phi_b2, bn_g2, bn_b2,
           feat_w, feat_b, gat_w, gat_b):
    layers = [
        (theta_w0, theta_b0, phi_w0, phi_b0, bn_g0, bn_b0),
        (theta_w1, theta_b1, phi_w1, phi_b1, bn_g1, bn_b1),
        (theta_w2, theta_b2, phi_w2, phi_b2, bn_g2, bn_b2),
    ]
    h3 = x
    hp = _pad128(x.reshape(BN, x.shape[-1]))
    for tw, tb, pw, pb, g, b in layers:
        idx0 = _knn_topk(h3, 0)
        g0 = _gather_h(idx0.reshape(B * NH * K), hp)
        idx1 = _knn_topk(h3, 1)
        g1 = _gather_h(idx1.reshape(B * NH * K), hp)
        m0, s0 = _edge_conv_max(g0, h3, tw, tb, pw, pb, 0)
        m1, s1 = _edge_conv_max(g1, h3, tw, tb, pw, pb, 1)
        h3, hp3 = _bn_final(m0, m1, s0, s1, g, b)
        hp = hp3.reshape(BN, 128)
    return _attn_pool(h3, gat_w, gat_b, feat_w, feat_b)


# speedup vs baseline: 1.0042x; 1.0042x over previous
import functools

import jax
import jax.numpy as jnp
from jax import lax
from jax.experimental import pallas as pl
from jax.experimental.pallas import tpu as pltpu
from jax.experimental.pallas import tpu_sc as plsc

B, N, K = 8, 2048, 16
BN = B * N
NH = N // 2


_RT = 256


def _knn_body(hr_ref, hf_ref, idx_ref):
    b = pl.program_id(0)
    hr = hr_ref[0]
    hf = hf_ref[0]
    sqf = jnp.sum(hf * hf, axis=1, keepdims=True)
    sqr = jnp.sum(hr * hr, axis=1, keepdims=True)
    g = lax.dot_general(hr, hf, (((1,), (1,)), ((), ())),
                        preferred_element_type=jnp.float32)
    d2 = sqr + sqf.T - 2.0 * g
    iof = lax.broadcasted_iota(jnp.int32, (_RT, 128), 1).astype(jnp.float32)
    ng = N // 128
    d2g = [d2[:, gi * 128:(gi + 1) * 128] for gi in range(ng)]
    cols = []
    amf = None
    for k in range(K):
        val = None
        idxf = iof
        for gi in range(ng):
            v = d2g[gi]
            gidx = iof + (128.0 * gi)
            if k > 0:
                v = jnp.where(gidx == amf, jnp.inf, v)
                d2g[gi] = v
            if gi == 0:
                val = v
            else:
                cond = v < val
                val = jnp.minimum(val, v)
                idxf = jnp.where(cond, gidx, idxf)
        m = jnp.min(val, axis=1, keepdims=True)
        amf = jnp.min(jnp.where(val == m, idxf, 3.0e9), axis=1,
                      keepdims=True)
        cols.append(amf.astype(jnp.int32))
    idx_ref[0] = jnp.concatenate(cols, axis=1) + b * N


def _knn_topk(h3, half):
    _, _, d = h3.shape
    nrt = NH // _RT
    grid = (B, nrt)
    return pl.pallas_call(
        _knn_body,
        grid=grid,
        in_specs=[
            pl.BlockSpec((1, _RT, d), lambda b, r: (b, r + half * nrt, 0)),
            pl.BlockSpec((1, N, d), lambda b, r: (b, 0, 0)),
        ],
        out_specs=pl.BlockSpec((1, _RT, K), lambda b, r: (b, r, 0)),
        out_shape=jax.ShapeDtypeStruct((B, NH, K), jnp.int32),
    )(h3, h3)


def _pad128_body(x_ref, out_ref):
    x = x_ref[...]
    out_ref[...] = jnp.concatenate(
        [x, jnp.zeros((x.shape[0], 128 - x.shape[1]), jnp.float32)], axis=1)


def _pad128(x2):
    M, d = x2.shape
    T = 1024
    return pl.pallas_call(
        _pad128_body,
        grid=(M // T,),
        in_specs=[pl.BlockSpec((T, d), lambda i: (i, 0))],
        out_specs=pl.BlockSpec((T, 128), lambda i: (i, 0)),
        out_shape=jax.ShapeDtypeStruct((M, 128), jnp.float32),
    )(x2)


_NC, _NS = 2, 16
_NW = _NC * _NS
_CH = 8


def _gather_h(idx_flat, hp):
    nidx = idx_flat.shape[0] // K
    npw = nidx // _NW
    nch = npw // _CH
    mesh = plsc.VectorSubcoreMesh(core_axis_name="c", subcore_axis_name="s")

    @functools.partial(
        pl.kernel,
        mesh=mesh,
        out_type=jax.ShapeDtypeStruct((nidx * K, 128), jnp.float32),
        scratch_types=[
            pltpu.VMEM((_CH * K,), jnp.int32),
            pltpu.VMEM((_CH * K,), jnp.int32),
            pltpu.VMEM((_CH * K, 128), jnp.float32),
            pltpu.VMEM((_CH * K, 128), jnp.float32),
            pltpu.SemaphoreType.DMA,
            pltpu.SemaphoreType.DMA,
        ],
    )
    def sc_k(idx_hbm, h_hbm, out_hbm,
             idx_v0, idx_v1, rows_v0, rows_v1, gs0, gs1):
        wid = lax.axis_index("s") * _NC + lax.axis_index("c")
        w0 = wid * npw * K
        idx_v = (idx_v0, idx_v1)
        rows_v = (rows_v0, rows_v1)
        gsem = (gs0, gs1)

        def start(ci, sl):
            base = w0 + ci * (_CH * K)
            pltpu.sync_copy(idx_hbm.at[pl.ds(base, _CH * K)], idx_v[sl])
            pltpu.async_copy(h_hbm.at[idx_v[sl]], rows_v[sl], gsem[sl])

        def drain(ci, sl):
            pltpu.make_async_copy(h_hbm.at[idx_v[sl]], rows_v[sl],
                                  gsem[sl]).wait()
            base = w0 + ci * (_CH * K)
            pltpu.sync_copy(rows_v[sl], out_hbm.at[pl.ds(base, _CH * K)])

        start(0, 0)

        def chunk_body(ci2, carry):
            c0 = ci2 * 2
            start(c0 + 1, 1)
            drain(c0, 0)

            @pl.when(c0 + 2 < nch)
            def _():
                start(c0 + 2, 0)

            drain(c0 + 1, 1)
            return carry

        lax.fori_loop(0, nch // 2, chunk_body, 0)

    return sc_k(idx_flat, hp)


_VT = 64


def _edge_conv_body(hnb_ref, h_ref, tw_ref, tb_ref, pw_ref, pb_ref,
                    maxe_ref, sums_ref, acc_ref):
    i = pl.program_id(0)
    j = pl.program_id(1)
    din = h_ref.shape[2]
    dout = tw_ref.shape[1]

    @pl.when((i == 0) & (j == 0))
    def _():
        acc_ref[...] = jnp.zeros_like(acc_ref)

    hn = hnb_ref[...][:, :din]
    hv = h_ref[0]
    hv_rep = jnp.broadcast_to(hv[:, None, :], (_VT, K, din))
    theta = hv_rep.reshape(_VT * K, din) - hn
    t1 = jnp.dot(theta, tw_ref[...], preferred_element_type=jnp.float32)
    phi = jnp.dot(hv, pw_ref[...],
                  preferred_element_type=jnp.float32) + pb_ref[0:1, :]
    e3 = (t1 + tb_ref[0:1, :]).reshape(_VT, K, dout) + phi[:, None, :]
    maxe_ref[0] = jnp.max(e3, axis=1)
    ef = e3.reshape(_VT * K, dout)
    acc_ref[0:1, :] += jnp.sum(ef, axis=0, keepdims=True)
    acc_ref[1:2, :] += jnp.sum(ef * ef, axis=0, keepdims=True)

    @pl.when((i == pl.num_programs(0) - 1) & (j == pl.num_programs(1) - 1))
    def _():
        sums_ref[...] = acc_ref[...]


def _edge_conv_max(hnb, h3, tw, tb, pw, pb, half):
    din, dout = tw.shape
    w = hnb.shape[1]
    nvt = NH // _VT
    grid = (B, nvt)
    return pl.pallas_call(
        _edge_conv_body,
        grid=grid,
        in_specs=[
            pl.BlockSpec((_VT * K, w), lambda b, j: (b * nvt + j, 0)),
            pl.BlockSpec((1, _VT, din),
                         lambda b, j: (b, j + half * nvt, 0)),
            pl.BlockSpec((din, dout), lambda b, j: (0, 0)),
            pl.BlockSpec((1, dout), lambda b, j: (0, 0)),
            pl.BlockSpec((din, dout), lambda b, j: (0, 0)),
            pl.BlockSpec((1, dout), lambda b, j: (0, 0)),
        ],
        out_specs=[
            pl.BlockSpec((1, _VT, dout), lambda b, j: (b, j, 0)),
            pl.BlockSpec((8, dout), lambda b, j: (0, 0)),
        ],
        out_shape=[
            jax.ShapeDtypeStruct((B, NH, dout), jnp.float32),
            jax.ShapeDtypeStruct((8, dout), jnp.float32),
        ],
        scratch_shapes=[pltpu.VMEM((8, dout), jnp.float32)],
    )(hnb, h3, tw, tb.reshape(1, dout), pw, pb.reshape(1, dout))


def _bn_final_body(x0_ref, x1_ref, s0_ref, s1_ref, g_ref, b_ref,
                   out_ref, pad_ref):
    E = float(BN * K)
    s0 = s0_ref[0:2, :] + s1_ref[0:2, :]
    mu = s0[0:1, :] / E
    var = jnp.maximum(s0[1:2, :] / E - mu * mu, 0.0)
    scale = g_ref[0:1, :]
    shift = b_ref[0:1, :]
    den = jnp.sqrt(var + 1e-5)

    def bn(x):
        t = (x - mu) / den * scale + shift
        return jnp.where(t >= 0, t, 0.2 * t)

    h0 = bn(x0_ref[0])
    h1 = bn(x1_ref[0])
    out_ref[0, :NH] = h0
    out_ref[0, NH:] = h1
    d = h0.shape[1]
    if d < 128:
        z = jnp.zeros((NH, 128 - d), jnp.float32)
        pad_ref[0, :NH] = jnp.concatenate([h0, z], axis=1)
        pad_ref[0, NH:] = jnp.concatenate([h1, z], axis=1)
    else:
        pad_ref[0, :NH] = h0
        pad_ref[0, NH:] = h1


def _bn_final(x0, x1, s0, s1, g, b):
    D = x0.shape[-1]
    return pl.pallas_call(
        _bn_final_body,
        grid=(B,),
        in_specs=[
            pl.BlockSpec((1, NH, D), lambda b: (b, 0, 0)),
            pl.BlockSpec((1, NH, D), lambda b: (b, 0, 0)),
            pl.BlockSpec((8, D), lambda b: (0, 0)),
            pl.BlockSpec((8, D), lambda b: (0, 0)),
            pl.BlockSpec((1, D), lambda b: (0, 0)),
            pl.BlockSpec((1, D), lambda b: (0, 0)),
        ],
        out_specs=[
            pl.BlockSpec((1, N, D), lambda b: (b, 0, 0)),
            pl.BlockSpec((1, N, 128), lambda b: (b, 0, 0)),
        ],
        out_shape=[
            jax.ShapeDtypeStruct((B, N, D), jnp.float32),
            jax.ShapeDtypeStruct((B, N, 128), jnp.float32),
        ],
    )(x0, x1, s0, s1, g.reshape(1, D), b.reshape(1, D))


def _pool_body(h_ref, gw_ref, gb_ref, fw_ref, fb_ref, out_ref):
    h = h_ref[0]
    g = jnp.dot(h, gw_ref[...],
                preferred_element_type=jnp.float32) + gb_ref[0:1, :]
    f = jnp.dot(h, fw_ref[...],
                preferred_element_type=jnp.float32) + fb_ref[0:1, :]
    g = jnp.maximum(g, 0.0)
    f = jnp.maximum(f, 0.0)
    mx = jnp.max(g, axis=0, keepdims=True)
    e = jnp.exp(g - mx)
    z = jnp.sum(e, axis=0, keepdims=True)
    out_ref[0] = jnp.sum(e * f, axis=0, keepdims=True) / z


def _attn_pool(h3, gw, gb, fw, fb):
    _, _, D = h3.shape
    DO = gw.shape[1]
    out = pl.pallas_call(
        _pool_body,
        grid=(B,),
        in_specs=[
            pl.BlockSpec((1, N, D), lambda b: (b, 0, 0)),
            pl.BlockSpec((D, DO), lambda b: (0, 0)),
            pl.BlockSpec((1, DO), lambda b: (0, 0)),
            pl.BlockSpec((D, DO), lambda b: (0, 0)),
            pl.BlockSpec((1, DO), lambda b: (0, 0)),
        ],
        out_specs=pl.BlockSpec((1, 1, DO), lambda b: (b, 0, 0)),
        out_shape=jax.ShapeDtypeStruct((B, 1, DO), jnp.float32),
    )(h3, gw, gb.reshape(1, DO), fw, fb.reshape(1, DO))
    return out.reshape(B, DO)


def kernel(x, theta_w0, theta_b0, phi_w0, phi_b0, bn_g0, bn_b0,
           theta_w1, theta_b1, phi_w1, phi_b1, bn_g1, bn_b1,
           theta_w2, theta_b2, phi_w2, phi_b2, bn_g2, bn_b2,
           feat_w, feat_b, gat_w, gat_b):
    layers = [
        (theta_w0, theta_b0, phi_w0, phi_b0, bn_g0, bn_b0),
        (theta_w1, theta_b1, phi_w1, phi_b1, bn_g1, bn_b1),
        (theta_w2, theta_b2, phi_w2, phi_b2, bn_g2, bn_b2),
    ]
    h3 = x
    hp = _pad128(x.reshape(BN, x.shape[-1]))
    for tw, tb, pw, pb, g, b in layers:
        idx0 = _knn_topk(h3, 0)
        g0 = _gather_h(idx0.reshape(B * NH * K), hp)
        idx1 = _knn_topk(h3, 1)
        g1 = _gather_h(idx1.reshape(B * NH * K), hp)
        m0, s0 = _edge_conv_max(g0, h3, tw, tb, pw, pb, 0)
        m1, s1 = _edge_conv_max(g1, h3, tw, tb, pw, pb, 1)
        h3, hp3 = _bn_final(m0, m1, s0, s1, g, b)
        hp = hp3.reshape(BN, 128)
    return _attn_pool(h3, gat_w, gat_b, feat_w, feat_b)


# speedup vs baseline: 1.1117x; 1.1071x over previous
import functools

import jax
import jax.numpy as jnp
from jax import lax
from jax.experimental import pallas as pl
from jax.experimental.pallas import tpu as pltpu
from jax.experimental.pallas import tpu_sc as plsc

B, N, K = 8, 2048, 16
BN = B * N
NH = N // 2


_RT = 512


def _knn_body(hr_ref, hf_ref, idx_ref):
    b = pl.program_id(0)
    hr = hr_ref[0]
    hf = hf_ref[0]
    sqf = jnp.sum(hf * hf, axis=1, keepdims=True)
    sqr = jnp.sum(hr * hr, axis=1, keepdims=True)
    g = lax.dot_general(hr, hf, (((1,), (1,)), ((), ())),
                        preferred_element_type=jnp.float32)
    d2 = sqr + sqf.T - 2.0 * g
    iof = lax.broadcasted_iota(jnp.int32, (_RT, 128), 1).astype(jnp.float32)
    ng = N // 128
    d2g = [d2[:, gi * 128:(gi + 1) * 128] for gi in range(ng)]
    cols = []
    amf = None
    for k in range(K):
        val = None
        idxf = iof
        for gi in range(ng):
            v = d2g[gi]
            gidx = iof + (128.0 * gi)
            if k > 0:
                v = jnp.where(gidx == amf, jnp.inf, v)
                d2g[gi] = v
            if gi == 0:
                val = v
            else:
                cond = v < val
                val = jnp.minimum(val, v)
                idxf = jnp.where(cond, gidx, idxf)
        m = jnp.min(val, axis=1, keepdims=True)
        amf = jnp.min(jnp.where(val == m, idxf, 3.0e9), axis=1,
                      keepdims=True)
        cols.append(amf.astype(jnp.int32))
    idx_ref[0] = jnp.concatenate(cols, axis=1) + b * N


def _knn_topk(h3, half):
    _, _, d = h3.shape
    nrt = NH // _RT
    grid = (B, nrt)
    return pl.pallas_call(
        _knn_body,
        grid=grid,
        in_specs=[
            pl.BlockSpec((1, _RT, d), lambda b, r: (b, r + half * nrt, 0)),
            pl.BlockSpec((1, N, d), lambda b, r: (b, 0, 0)),
        ],
        out_specs=pl.BlockSpec((1, _RT, K), lambda b, r: (b, r, 0)),
        out_shape=jax.ShapeDtypeStruct((B, NH, K), jnp.int32),
    )(h3, h3)


def _pad128_body(x_ref, out_ref):
    x = x_ref[...]
    out_ref[...] = jnp.concatenate(
        [x, jnp.zeros((x.shape[0], 128 - x.shape[1]), jnp.float32)], axis=1)


def _pad128(x2):
    M, d = x2.shape
    T = 1024
    return pl.pallas_call(
        _pad128_body,
        grid=(M // T,),
        in_specs=[pl.BlockSpec((T, d), lambda i: (i, 0))],
        out_specs=pl.BlockSpec((T, 128), lambda i: (i, 0)),
        out_shape=jax.ShapeDtypeStruct((M, 128), jnp.float32),
    )(x2)


_NC, _NS = 2, 16
_NW = _NC * _NS
_CH = 8


def _gather_h(idx_flat, hp):
    nidx = idx_flat.shape[0] // K
    npw = nidx // _NW
    nch = npw // _CH
    mesh = plsc.VectorSubcoreMesh(core_axis_name="c", subcore_axis_name="s")

    @functools.partial(
        pl.kernel,
        mesh=mesh,
        out_type=jax.ShapeDtypeStruct((nidx * K, 128), jnp.float32),
        scratch_types=[
            pltpu.VMEM((npw * K,), jnp.int32),
            pltpu.VMEM((_CH * K, 128), jnp.float32),
            pltpu.VMEM((_CH * K, 128), jnp.float32),
            pltpu.SemaphoreType.DMA,
            pltpu.SemaphoreType.DMA,
        ],
    )
    def sc_k(idx_hbm, h_hbm, out_hbm,
             idx_all, rows_v0, rows_v1, gs0, gs1):
        wid = lax.axis_index("s") * _NC + lax.axis_index("c")
        w0 = wid * npw * K
        rows_v = (rows_v0, rows_v1)
        gsem = (gs0, gs1)
        pltpu.sync_copy(idx_hbm.at[pl.ds(w0, npw * K)], idx_all)

        def start(ci, sl):
            pltpu.async_copy(
                h_hbm.at[idx_all.at[pl.ds(ci * (_CH * K), _CH * K)]],
                rows_v[sl], gsem[sl])

        def drain(ci, sl):
            pltpu.make_async_copy(
                h_hbm.at[idx_all.at[pl.ds(ci * (_CH * K), _CH * K)]],
                rows_v[sl], gsem[sl]).wait()
            base = w0 + ci * (_CH * K)
            pltpu.sync_copy(rows_v[sl], out_hbm.at[pl.ds(base, _CH * K)])

        start(0, 0)

        def chunk_body(ci2, carry):
            c0 = ci2 * 2
            start(c0 + 1, 1)
            drain(c0, 0)

            @pl.when(c0 + 2 < nch)
            def _():
                start(c0 + 2, 0)

            drain(c0 + 1, 1)
            return carry

        lax.fori_loop(0, nch // 2, chunk_body, 0)

    return sc_k(idx_flat, hp)


_VT = 64


def _edge_conv_body(hnb_ref, h_ref, tw_ref, tb_ref, pw_ref, pb_ref,
                    maxe_ref, sums_ref, acc_ref):
    i = pl.program_id(0)
    j = pl.program_id(1)
    din = h_ref.shape[2]
    dout = tw_ref.shape[1]

    @pl.when((i == 0) & (j == 0))
    def _():
        acc_ref[...] = jnp.zeros_like(acc_ref)

    hn = hnb_ref[...][:, :din]
    hv = h_ref[0]
    hv_rep = jnp.broadcast_to(hv[:, None, :], (_VT, K, din))
    theta = hv_rep.reshape(_VT * K, din) - hn
    t1 = jnp.dot(theta, tw_ref[...], preferred_element_type=jnp.float32)
    phi = jnp.dot(hv, pw_ref[...],
                  preferred_element_type=jnp.float32) + pb_ref[0:1, :]
    e3 = (t1 + tb_ref[0:1, :]).reshape(_VT, K, dout) + phi[:, None, :]
    maxe_ref[0] = jnp.max(e3, axis=1)
    ef = e3.reshape(_VT * K, dout)
    acc_ref[0:1, :] += jnp.sum(ef, axis=0, keepdims=True)
    acc_ref[1:2, :] += jnp.sum(ef * ef, axis=0, keepdims=True)

    @pl.when((i == pl.num_programs(0) - 1) & (j == pl.num_programs(1) - 1))
    def _():
        sums_ref[...] = acc_ref[...]


def _edge_conv_max(hnb, h3, tw, tb, pw, pb, half):
    din, dout = tw.shape
    w = hnb.shape[1]
    nvt = NH // _VT
    grid = (B, nvt)
    return pl.pallas_call(
        _edge_conv_body,
        grid=grid,
        in_specs=[
            pl.BlockSpec((_VT * K, w), lambda b, j: (b * nvt + j, 0)),
            pl.BlockSpec((1, _VT, din),
                         lambda b, j: (b, j + half * nvt, 0)),
            pl.BlockSpec((din, dout), lambda b, j: (0, 0)),
            pl.BlockSpec((1, dout), lambda b, j: (0, 0)),
            pl.BlockSpec((din, dout), lambda b, j: (0, 0)),
            pl.BlockSpec((1, dout), lambda b, j: (0, 0)),
        ],
        out_specs=[
            pl.BlockSpec((1, _VT, dout), lambda b, j: (b, j, 0)),
            pl.BlockSpec((8, dout), lambda b, j: (0, 0)),
        ],
        out_shape=[
            jax.ShapeDtypeStruct((B, NH, dout), jnp.float32),
            jax.ShapeDtypeStruct((8, dout), jnp.float32),
        ],
        scratch_shapes=[pltpu.VMEM((8, dout), jnp.float32)],
    )(hnb, h3, tw, tb.reshape(1, dout), pw, pb.reshape(1, dout))


def _bn_final_body(x0_ref, x1_ref, s0_ref, s1_ref, g_ref, b_ref,
                   out_ref, pad_ref):
    E = float(BN * K)
    s0 = s0_ref[0:2, :] + s1_ref[0:2, :]
    mu = s0[0:1, :] / E
    var = jnp.maximum(s0[1:2, :] / E - mu * mu, 0.0)
    scale = g_ref[0:1, :]
    shift = b_ref[0:1, :]
    den = jnp.sqrt(var + 1e-5)

    def bn(x):
        t = (x - mu) / den * scale + shift
        return jnp.where(t >= 0, t, 0.2 * t)

    h0 = bn(x0_ref[0])
    h1 = bn(x1_ref[0])
    out_ref[0, :NH] = h0
    out_ref[0, NH:] = h1
    d = h0.shape[1]
    if d < 128:
        z = jnp.zeros((NH, 128 - d), jnp.float32)
        pad_ref[0, :NH] = jnp.concatenate([h0, z], axis=1)
        pad_ref[0, NH:] = jnp.concatenate([h1, z], axis=1)
    else:
        pad_ref[0, :NH] = h0
        pad_ref[0, NH:] = h1


def _bn_final(x0, x1, s0, s1, g, b):
    D = x0.shape[-1]
    return pl.pallas_call(
        _bn_final_body,
        grid=(B,),
        in_specs=[
            pl.BlockSpec((1, NH, D), lambda b: (b, 0, 0)),
            pl.BlockSpec((1, NH, D), lambda b: (b, 0, 0)),
            pl.BlockSpec((8, D), lambda b: (0, 0)),
            pl.BlockSpec((8, D), lambda b: (0, 0)),
            pl.BlockSpec((1, D), lambda b: (0, 0)),
            pl.BlockSpec((1, D), lambda b: (0, 0)),
        ],
        out_specs=[
            pl.BlockSpec((1, N, D), lambda b: (b, 0, 0)),
            pl.BlockSpec((1, N, 128), lambda b: (b, 0, 0)),
        ],
        out_shape=[
            jax.ShapeDtypeStruct((B, N, D), jnp.float32),
            jax.ShapeDtypeStruct((B, N, 128), jnp.float32),
        ],
    )(x0, x1, s0, s1, g.reshape(1, D), b.reshape(1, D))


def _pool_body(h_ref, gw_ref, gb_ref, fw_ref, fb_ref, out_ref):
    h = h_ref[0]
    g = jnp.dot(h, gw_ref[...],
                preferred_element_type=jnp.float32) + gb_ref[0:1, :]
    f = jnp.dot(h, fw_ref[...],
                preferred_element_type=jnp.float32) + fb_ref[0:1, :]
    g = jnp.maximum(g, 0.0)
    f = jnp.maximum(f, 0.0)
    mx = jnp.max(g, axis=0, keepdims=True)
    e = jnp.exp(g - mx)
    z = jnp.sum(e, axis=0, keepdims=True)
    out_ref[0] = jnp.sum(e * f, axis=0, keepdims=True) / z


def _attn_pool(h3, gw, gb, fw, fb):
    _, _, D = h3.shape
    DO = gw.shape[1]
    out = pl.pallas_call(
        _pool_body,
        grid=(B,),
        in_specs=[
            pl.BlockSpec((1, N, D), lambda b: (b, 0, 0)),
            pl.BlockSpec((D, DO), lambda b: (0, 0)),
            pl.BlockSpec((1, DO), lambda b: (0, 0)),
            pl.BlockSpec((D, DO), lambda b: (0, 0)),
            pl.BlockSpec((1, DO), lambda b: (0, 0)),
        ],
        out_specs=pl.BlockSpec((1, 1, DO), lambda b: (b, 0, 0)),
        out_shape=jax.ShapeDtypeStruct((B, 1, DO), jnp.float32),
    )(h3, gw, gb.reshape(1, DO), fw, fb.reshape(1, DO))
    return out.reshape(B, DO)


def kernel(x, theta_w0, theta_b0, phi_w0, phi_b0, bn_g0, bn_b0,
           theta_w1, theta_b1, phi_w1, phi_b1, bn_g1, bn_b1,
           theta_w2, theta_b2, phi_w2, phi_b2, bn_g2, bn_b2,
           feat_w, feat_b, gat_w, gat_b):
    layers = [
        (theta_w0, theta_b0, phi_w0, phi_b0, bn_g0, bn_b0),
        (theta_w1, theta_b1, phi_w1, phi_b1, bn_g1, bn_b1),
        (theta_w2, theta_b2, phi_w2, phi_b2, bn_g2, bn_b2),
    ]
    h3 = x
    hp = _pad128(x.reshape(BN, x.shape[-1]))
    for tw, tb, pw, pb, g, b in layers:
        idx0 = _knn_topk(h3, 0)
        g0 = _gather_h(idx0.reshape(B * NH * K), hp)
        idx1 = _knn_topk(h3, 1)
        g1 = _gather_h(idx1.reshape(B * NH * K), hp)
        m0, s0 = _edge_conv_max(g0, h3, tw, tb, pw, pb, 0)
        m1, s1 = _edge_conv_max(g1, h3, tw, tb, pw, pb, 1)
        h3, hp3 = _bn_final(m0, m1, s0, s1, g, b)
        hp = hp3.reshape(BN, 128)
    return _attn_pool(h3, gat_w, gat_b, feat_w, feat_b)


# speedup vs baseline: 1.1145x; 1.0025x over previous
import functools

import jax
import jax.numpy as jnp
from jax import lax
from jax.experimental import pallas as pl
from jax.experimental.pallas import tpu as pltpu
from jax.experimental.pallas import tpu_sc as plsc

B, N, K = 8, 2048, 16
BN = B * N
NH = N // 2


_RT = 1024


def _knn_body(hr_ref, hf_ref, idx_ref):
    b = pl.program_id(0)
    hr = hr_ref[0]
    hf = hf_ref[0]
    sqf = jnp.sum(hf * hf, axis=1, keepdims=True)
    sqr = jnp.sum(hr * hr, axis=1, keepdims=True)
    g = lax.dot_general(hr, hf, (((1,), (1,)), ((), ())),
                        preferred_element_type=jnp.float32)
    d2 = sqr + sqf.T - 2.0 * g
    iof = lax.broadcasted_iota(jnp.int32, (_RT, 128), 1).astype(jnp.float32)
    ng = N // 128
    d2g = [d2[:, gi * 128:(gi + 1) * 128] for gi in range(ng)]
    cols = []
    amf = None
    for k in range(K):
        val = None
        idxf = iof
        for gi in range(ng):
            v = d2g[gi]
            gidx = iof + (128.0 * gi)
            if k > 0:
                v = jnp.where(gidx == amf, jnp.inf, v)
                d2g[gi] = v
            if gi == 0:
                val = v
            else:
                cond = v < val
                val = jnp.minimum(val, v)
                idxf = jnp.where(cond, gidx, idxf)
        m = jnp.min(val, axis=1, keepdims=True)
        amf = jnp.min(jnp.where(val == m, idxf, 3.0e9), axis=1,
                      keepdims=True)
        cols.append(amf.astype(jnp.int32))
    idx_ref[0] = jnp.concatenate(cols, axis=1) + b * N


def _knn_topk(h3, half):
    _, _, d = h3.shape
    nrt = NH // _RT
    grid = (B, nrt)
    return pl.pallas_call(
        _knn_body,
        grid=grid,
        in_specs=[
            pl.BlockSpec((1, _RT, d), lambda b, r: (b, r + half * nrt, 0)),
            pl.BlockSpec((1, N, d), lambda b, r: (b, 0, 0)),
        ],
        out_specs=pl.BlockSpec((1, _RT, K), lambda b, r: (b, r, 0)),
        out_shape=jax.ShapeDtypeStruct((B, NH, K), jnp.int32),
    )(h3, h3)


def _pad128_body(x_ref, out_ref):
    x = x_ref[...]
    out_ref[...] = jnp.concatenate(
        [x, jnp.zeros((x.shape[0], 128 - x.shape[1]), jnp.float32)], axis=1)


def _pad128(x2):
    M, d = x2.shape
    T = 1024
    return pl.pallas_call(
        _pad128_body,
        grid=(M // T,),
        in_specs=[pl.BlockSpec((T, d), lambda i: (i, 0))],
        out_specs=pl.BlockSpec((T, 128), lambda i: (i, 0)),
        out_shape=jax.ShapeDtypeStruct((M, 128), jnp.float32),
    )(x2)


_NC, _NS = 2, 16
_NW = _NC * _NS
_CH = 8


def _gather_h(idx_flat, hp):
    nidx = idx_flat.shape[0] // K
    npw = nidx // _NW
    nch = npw // _CH
    mesh = plsc.VectorSubcoreMesh(core_axis_name="c", subcore_axis_name="s")

    @functools.partial(
        pl.kernel,
        mesh=mesh,
        out_type=jax.ShapeDtypeStruct((nidx * K, 128), jnp.float32),
        scratch_types=[
            pltpu.VMEM((npw * K,), jnp.int32),
            pltpu.VMEM((_CH * K, 128), jnp.float32),
            pltpu.VMEM((_CH * K, 128), jnp.float32),
            pltpu.SemaphoreType.DMA,
            pltpu.SemaphoreType.DMA,
        ],
    )
    def sc_k(idx_hbm, h_hbm, out_hbm,
             idx_all, rows_v0, rows_v1, gs0, gs1):
        wid = lax.axis_index("s") * _NC + lax.axis_index("c")
        w0 = wid * npw * K
        rows_v = (rows_v0, rows_v1)
        gsem = (gs0, gs1)
        pltpu.sync_copy(idx_hbm.at[pl.ds(w0, npw * K)], idx_all)

        def start(ci, sl):
            pltpu.async_copy(
                h_hbm.at[idx_all.at[pl.ds(ci * (_CH * K), _CH * K)]],
                rows_v[sl], gsem[sl])

        def drain(ci, sl):
            pltpu.make_async_copy(
                h_hbm.at[idx_all.at[pl.ds(ci * (_CH * K), _CH * K)]],
                rows_v[sl], gsem[sl]).wait()
            base = w0 + ci * (_CH * K)
            pltpu.sync_copy(rows_v[sl], out_hbm.at[pl.ds(base, _CH * K)])

        start(0, 0)

        def chunk_body(ci2, carry):
            c0 = ci2 * 2
            start(c0 + 1, 1)
            drain(c0, 0)

            @pl.when(c0 + 2 < nch)
            def _():
                start(c0 + 2, 0)

            drain(c0 + 1, 1)
            return carry

        lax.fori_loop(0, nch // 2, chunk_body, 0)

    return sc_k(idx_flat, hp)


_VT = 128


def _edge_conv_body(hnb_ref, h_ref, tw_ref, tb_ref, pw_ref, pb_ref,
                    maxe_ref, sums_ref, acc_ref):
    i = pl.program_id(0)
    j = pl.program_id(1)
    din = h_ref.shape[2]
    dout = tw_ref.shape[1]

    @pl.when((i == 0) & (j == 0))
    def _():
        acc_ref[...] = jnp.zeros_like(acc_ref)

    hn = hnb_ref[...][:, :din]
    hv = h_ref[0]
    hv_rep = jnp.broadcast_to(hv[:, None, :], (_VT, K, din))
    theta = hv_rep.reshape(_VT * K, din) - hn
    t1 = jnp.dot(theta, tw_ref[...], preferred_element_type=jnp.float32)
    phi = jnp.dot(hv, pw_ref[...],
                  preferred_element_type=jnp.float32) + pb_ref[0:1, :]
    e3 = (t1 + tb_ref[0:1, :]).reshape(_VT, K, dout) + phi[:, None, :]
    maxe_ref[0] = jnp.max(e3, axis=1)
    ef = e3.reshape(_VT * K, dout)
    acc_ref[0:1, :] += jnp.sum(ef, axis=0, keepdims=True)
    acc_ref[1:2, :] += jnp.sum(ef * ef, axis=0, keepdims=True)

    @pl.when((i == pl.num_programs(0) - 1) & (j == pl.num_programs(1) - 1))
    def _():
        sums_ref[...] = acc_ref[...]


def _edge_conv_max(hnb, h3, tw, tb, pw, pb, half):
    din, dout = tw.shape
    w = hnb.shape[1]
    nvt = NH // _VT
    grid = (B, nvt)
    return pl.pallas_call(
        _edge_conv_body,
        grid=grid,
        in_specs=[
            pl.BlockSpec((_VT * K, w), lambda b, j: (b * nvt + j, 0)),
            pl.BlockSpec((1, _VT, din),
                         lambda b, j: (b, j + half * nvt, 0)),
            pl.BlockSpec((din, dout), lambda b, j: (0, 0)),
            pl.BlockSpec((1, dout), lambda b, j: (0, 0)),
            pl.BlockSpec((din, dout), lambda b, j: (0, 0)),
            pl.BlockSpec((1, dout), lambda b, j: (0, 0)),
        ],
        out_specs=[
            pl.BlockSpec((1, _VT, dout), lambda b, j: (b, j, 0)),
            pl.BlockSpec((8, dout), lambda b, j: (0, 0)),
        ],
        out_shape=[
            jax.ShapeDtypeStruct((B, NH, dout), jnp.float32),
            jax.ShapeDtypeStruct((8, dout), jnp.float32),
        ],
        scratch_shapes=[pltpu.VMEM((8, dout), jnp.float32)],
    )(hnb, h3, tw, tb.reshape(1, dout), pw, pb.reshape(1, dout))


def _bn_final_body(x0_ref, x1_ref, s0_ref, s1_ref, g_ref, b_ref,
                   out_ref, pad_ref):
    E = float(BN * K)
    s0 = s0_ref[0:2, :] + s1_ref[0:2, :]
    mu = s0[0:1, :] / E
    var = jnp.maximum(s0[1:2, :] / E - mu * mu, 0.0)
    scale = g_ref[0:1, :]
    shift = b_ref[0:1, :]
    den = jnp.sqrt(var + 1e-5)

    def bn(x):
        t = (x - mu) / den * scale + shift
        return jnp.where(t >= 0, t, 0.2 * t)

    h0 = bn(x0_ref[0])
    h1 = bn(x1_ref[0])
    out_ref[0, :NH] = h0
    out_ref[0, NH:] = h1
    d = h0.shape[1]
    if d < 128:
        z = jnp.zeros((NH, 128 - d), jnp.float32)
        pad_ref[0, :NH] = jnp.concatenate([h0, z], axis=1)
        pad_ref[0, NH:] = jnp.concatenate([h1, z], axis=1)
    else:
        pad_ref[0, :NH] = h0
        pad_ref[0, NH:] = h1


def _bn_final(x0, x1, s0, s1, g, b):
    D = x0.shape[-1]
    return pl.pallas_call(
        _bn_final_body,
        grid=(B,),
        in_specs=[
            pl.BlockSpec((1, NH, D), lambda b: (b, 0, 0)),
            pl.BlockSpec((1, NH, D), lambda b: (b, 0, 0)),
            pl.BlockSpec((8, D), lambda b: (0, 0)),
            pl.BlockSpec((8, D), lambda b: (0, 0)),
            pl.BlockSpec((1, D), lambda b: (0, 0)),
            pl.BlockSpec((1, D), lambda b: (0, 0)),
        ],
        out_specs=[
            pl.BlockSpec((1, N, D), lambda b: (b, 0, 0)),
            pl.BlockSpec((1, N, 128), lambda b: (b, 0, 0)),
        ],
        out_shape=[
            jax.ShapeDtypeStruct((B, N, D), jnp.float32),
            jax.ShapeDtypeStruct((B, N, 128), jnp.float32),
        ],
    )(x0, x1, s0, s1, g.reshape(1, D), b.reshape(1, D))


def _pool_body(h_ref, gw_ref, gb_ref, fw_ref, fb_ref, out_ref):
    h = h_ref[0]
    g = jnp.dot(h, gw_ref[...],
                preferred_element_type=jnp.float32) + gb_ref[0:1, :]
    f = jnp.dot(h, fw_ref[...],
                preferred_element_type=jnp.float32) + fb_ref[0:1, :]
    g = jnp.maximum(g, 0.0)
    f = jnp.maximum(f, 0.0)
    mx = jnp.max(g, axis=0, keepdims=True)
    e = jnp.exp(g - mx)
    z = jnp.sum(e, axis=0, keepdims=True)
    out_ref[0] = jnp.sum(e * f, axis=0, keepdims=True) / z


def _attn_pool(h3, gw, gb, fw, fb):
    _, _, D = h3.shape
    DO = gw.shape[1]
    out = pl.pallas_call(
        _pool_body,
        grid=(B,),
        in_specs=[
            pl.BlockSpec((1, N, D), lambda b: (b, 0, 0)),
            pl.BlockSpec((D, DO), lambda b: (0, 0)),
            pl.BlockSpec((1, DO), lambda b: (0, 0)),
            pl.BlockSpec((D, DO), lambda b: (0, 0)),
            pl.BlockSpec((1, DO), lambda b: (0, 0)),
        ],
        out_specs=pl.BlockSpec((1, 1, DO), lambda b: (b, 0, 0)),
        out_shape=jax.ShapeDtypeStruct((B, 1, DO), jnp.float32),
    )(h3, gw, gb.reshape(1, DO), fw, fb.reshape(1, DO))
    return out.reshape(B, DO)


def kernel(x, theta_w0, theta_b0, phi_w0, phi_b0, bn_g0, bn_b0,
           theta_w1, theta_b1, phi_w1, phi_b1, bn_g1, bn_b1,
           theta_w2, theta_b2, phi_w2, phi_b2, bn_g2, bn_b2,
           feat_w, feat_b, gat_w, gat_b):
    layers = [
        (theta_w0, theta_b0, phi_w0, phi_b0, bn_g0, bn_b0),
        (theta_w1, theta_b1, phi_w1, phi_b1, bn_g1, bn_b1),
        (theta_w2, theta_b2, phi_w2, phi_b2, bn_g2, bn_b2),
    ]
    h3 = x
    hp = _pad128(x.reshape(BN, x.shape[-1]))
    for tw, tb, pw, pb, g, b in layers:
        idx0 = _knn_topk(h3, 0)
        g0 = _gather_h(idx0.reshape(B * NH * K), hp)
        idx1 = _knn_topk(h3, 1)
        g1 = _gather_h(idx1.reshape(B * NH * K), hp)
        m0, s0 = _edge_conv_max(g0, h3, tw, tb, pw, pb, 0)
        m1, s1 = _edge_conv_max(g1, h3, tw, tb, pw, pb, 1)
        h3, hp3 = _bn_final(m0, m1, s0, s1, g, b)
        hp = hp3.reshape(BN, 128)
    return _attn_pool(h3, gat_w, gat_b, feat_w, feat_b)
